# baseline (device time: 20623 ns/iter reference)
import jax
import jax.numpy as jnp
from jax import lax
from jax.experimental import pallas as pl
from jax.experimental.pallas import tpu as pltpu

_DIMS = (((1,), (0,)), ((), ()))
_C = 16


def kernel(x, dy):
    m, d = x.shape
    _, f = dy.shape
    half = d // 2
    zh = half // 2
    fc = f // _C

    def body(x_ref, dy_ref, out_ref, dyv, xt, ysend, yrecv, zsend, zrecv,
             zredf, zof, dy_sem, out_sem,
             ysend_sem, yrecv_sem, zsend_sem, zrecv_sem):
        my_x = lax.axis_index("x")
        my_y = lax.axis_index("y")
        my_z = lax.axis_index("z")
        ypartner = (my_x, 1 - my_y, my_z)
        zpartner = (my_x, my_y, 1 - my_z)

        dy_cp = pltpu.make_async_copy(dy_ref, dyv, dy_sem)
        dy_cp.start()

        barrier_sem = pltpu.get_barrier_semaphore()
        for nbr in (ypartner, zpartner):
            pl.semaphore_signal(
                barrier_sem, inc=1, device_id=nbr,
                device_id_type=pl.DeviceIdType.MESH,
            )
        pl.semaphore_wait(barrier_sem, 2)

        xt[0] = x_ref[:, pl.ds((1 - my_y) * half + my_z * zh, zh)].T
        xt[1] = x_ref[:, pl.ds(my_y * half + my_z * zh, zh)].T
        xs = xt[0]
        xo = xt[1]

        dy_cp.wait()

        y_rdmas = []
        for i in range(_C):
            ps = lax.dot_general(
                xs, dyv[:, i * fc:(i + 1) * fc], _DIMS,
                preferred_element_type=jnp.float32,
            )
            ysend[i] = ps.astype(jnp.bfloat16)
            r = pltpu.make_async_remote_copy(
                src_ref=ysend.at[i], dst_ref=yrecv.at[i],
                send_sem=ysend_sem.at[i], recv_sem=yrecv_sem.at[i],
                device_id=ypartner, device_id_type=pl.DeviceIdType.MESH,
            )
            r.start()
            y_rdmas.append(r)

        z_rdmas = []
        out_cps = []
        for i in range(_C):
            own = lax.dot_general(
                xo, dyv[:, i * fc:(i + 1) * fc], _DIMS,
                preferred_element_type=jnp.float32,
            )
            y_rdmas[i].wait_recv()
            red = own + yrecv[i].astype(jnp.float32)
            zredf[i] = red
            zsend[i] = red.astype(jnp.bfloat16)
            r = pltpu.make_async_remote_copy(
                src_ref=zsend.at[i], dst_ref=zrecv.at[i],
                send_sem=zsend_sem.at[i], recv_sem=zrecv_sem.at[i],
                device_id=zpartner, device_id_type=pl.DeviceIdType.MESH,
            )
            r.start()
            z_rdmas.append(r)
            cp = pltpu.make_async_copy(
                zredf.at[i],
                out_ref.at[pl.ds(my_z * zh, zh), pl.ds(i * fc, fc)],
                out_sem.at[i],
            )
            cp.start()
            out_cps.append(cp)

        for i in range(_C):
            z_rdmas[i].wait_recv()
            zof[i] = zrecv[i].astype(jnp.float32)
            cp = pltpu.make_async_copy(
                zof.at[i],
                out_ref.at[pl.ds((1 - my_z) * zh, zh), pl.ds(i * fc, fc)],
                out_sem.at[_C + i],
            )
            cp.start()
            out_cps.append(cp)

        for cp in out_cps:
            cp.wait()
        for i in range(_C):
            y_rdmas[i].wait_send()
            z_rdmas[i].wait_send()

    return pl.pallas_call(
        body,
        out_shape=jax.ShapeDtypeStruct((half, f), jnp.float32),
        in_specs=[
            pl.BlockSpec(memory_space=pltpu.VMEM),
            pl.BlockSpec(memory_space=pltpu.MemorySpace.HBM),
        ],
        out_specs=pl.BlockSpec(memory_space=pltpu.MemorySpace.HBM),
        scratch_shapes=[
            pltpu.VMEM((m, f), jnp.float32),
            pltpu.VMEM((2, zh, m), jnp.float32),
            pltpu.VMEM((_C, zh, fc), jnp.bfloat16),
            pltpu.VMEM((_C, zh, fc), jnp.bfloat16),
            pltpu.VMEM((_C, zh, fc), jnp.bfloat16),
            pltpu.VMEM((_C, zh, fc), jnp.bfloat16),
            pltpu.VMEM((_C, zh, fc), jnp.float32),
            pltpu.VMEM((_C, zh, fc), jnp.float32),
            pltpu.SemaphoreType.DMA,
            pltpu.SemaphoreType.DMA((2 * _C,)),
            pltpu.SemaphoreType.DMA((_C,)),
            pltpu.SemaphoreType.DMA((_C,)),
            pltpu.SemaphoreType.DMA((_C,)),
            pltpu.SemaphoreType.DMA((_C,)),
        ],
        compiler_params=pltpu.CompilerParams(collective_id=0),
    )(x, dy)


# device time: 19957 ns/iter; 1.0334x vs baseline; 1.0334x over previous
import jax
import jax.numpy as jnp
from jax import lax
from jax.experimental import pallas as pl
from jax.experimental.pallas import tpu as pltpu

_DIMS = (((1,), (0,)), ((), ()))
_C = 8


def kernel(x, dy):
    m, d = x.shape
    _, f = dy.shape
    half = d // 2
    qh = half // 4
    fc = f // _C

    def body(x_ref, dy_ref, out_ref, dyv, xt, ysend, yrecv, agb,
             xrecv, zrecv, frecv, qredf, xof, zof, fof,
             dy_sem, out_sem, ysend_sem, yrecv_sem,
             xsend_sem, xrecv_sem, zsend_sem, zrecv_sem,
             fsend_sem, frecv_sem):
        my_x = lax.axis_index("x")
        my_y = lax.axis_index("y")
        my_z = lax.axis_index("z")
        ypartner = (my_x, 1 - my_y, my_z)
        xpartner = (1 - my_x, my_y, my_z)
        zpartner = (my_x, my_y, 1 - my_z)

        def qrow(xi, zi):
            return zi * (2 * qh) + xi * qh

        dy_cp = pltpu.make_async_copy(dy_ref, dyv, dy_sem)
        dy_cp.start()

        barrier_sem = pltpu.get_barrier_semaphore()
        for nbr in (ypartner, xpartner, zpartner):
            pl.semaphore_signal(
                barrier_sem, inc=1, device_id=nbr,
                device_id_type=pl.DeviceIdType.MESH,
            )
        pl.semaphore_wait(barrier_sem, 3)

        myq = qrow(my_x, my_z)
        zbase = my_z * 2 * qh
        xt[0] = x_ref[:, pl.ds((1 - my_y) * half + zbase, 2 * qh)].T
        xt[1] = x_ref[:, pl.ds(my_y * half + zbase, 2 * qh)].T
        xs = xt[0, pl.ds(my_x * qh, qh), :]
        xo = xt[1, pl.ds(my_x * qh, qh), :]

        dy_cp.wait()

        y_rdmas = []
        for i in range(_C):
            ps = lax.dot_general(
                xs, dyv[:, i * fc:(i + 1) * fc], _DIMS,
                preferred_element_type=jnp.float32,
            )
            ysend[i] = ps.astype(jnp.bfloat16)
            r = pltpu.make_async_remote_copy(
                src_ref=ysend.at[i], dst_ref=yrecv.at[i],
                send_sem=ysend_sem.at[i], recv_sem=yrecv_sem.at[i],
                device_id=ypartner, device_id_type=pl.DeviceIdType.MESH,
            )
            r.start()
            y_rdmas.append(r)

        x_rdmas, z_rdmas, f_rdmas, out_cps = [], [], [], []
        for i in range(_C):
            own = lax.dot_general(
                xo, dyv[:, i * fc:(i + 1) * fc], _DIMS,
                preferred_element_type=jnp.float32,
            )
            y_rdmas[i].wait_recv()
            red = own + yrecv[i].astype(jnp.float32)
            qredf[i] = red
            agb[i] = red.astype(jnp.bfloat16)
            rx = pltpu.make_async_remote_copy(
                src_ref=agb.at[i], dst_ref=xrecv.at[i],
                send_sem=xsend_sem.at[i], recv_sem=xrecv_sem.at[i],
                device_id=xpartner, device_id_type=pl.DeviceIdType.MESH,
            )
            rx.start()
            x_rdmas.append(rx)
            rz = pltpu.make_async_remote_copy(
                src_ref=agb.at[i], dst_ref=zrecv.at[i],
                send_sem=zsend_sem.at[i], recv_sem=zrecv_sem.at[i],
                device_id=zpartner, device_id_type=pl.DeviceIdType.MESH,
            )
            rz.start()
            z_rdmas.append(rz)
            cp = pltpu.make_async_copy(
                qredf.at[i],
                out_ref.at[pl.ds(myq, qh), pl.ds(i * fc, fc)],
                out_sem.at[i],
            )
            cp.start()
            out_cps.append(cp)

        for i in range(_C):
            x_rdmas[i].wait_recv()
            rf = pltpu.make_async_remote_copy(
                src_ref=xrecv.at[i], dst_ref=frecv.at[i],
                send_sem=fsend_sem.at[i], recv_sem=frecv_sem.at[i],
                device_id=zpartner, device_id_type=pl.DeviceIdType.MESH,
            )
            rf.start()
            f_rdmas.append(rf)
            xof[i] = xrecv[i].astype(jnp.float32)
            cp = pltpu.make_async_copy(
                xof.at[i],
                out_ref.at[pl.ds(qrow(1 - my_x, my_z), qh), pl.ds(i * fc, fc)],
                out_sem.at[_C + i],
            )
            cp.start()
            out_cps.append(cp)

        for i in range(_C):
            z_rdmas[i].wait_recv()
            zof[i] = zrecv[i].astype(jnp.float32)
            cp = pltpu.make_async_copy(
                zof.at[i],
                out_ref.at[pl.ds(qrow(my_x, 1 - my_z), qh), pl.ds(i * fc, fc)],
                out_sem.at[2 * _C + i],
            )
            cp.start()
            out_cps.append(cp)

        for i in range(_C):
            f_rdmas[i].wait_recv()
            fof[i] = frecv[i].astype(jnp.float32)
            cp = pltpu.make_async_copy(
                fof.at[i],
                out_ref.at[
                    pl.ds(qrow(1 - my_x, 1 - my_z), qh), pl.ds(i * fc, fc)
                ],
                out_sem.at[3 * _C + i],
            )
            cp.start()
            out_cps.append(cp)

        for cp in out_cps:
            cp.wait()
        for i in range(_C):
            y_rdmas[i].wait_send()
            x_rdmas[i].wait_send()
            z_rdmas[i].wait_send()
            f_rdmas[i].wait_send()

    qshape = (_C, qh, fc)
    return pl.pallas_call(
        body,
        out_shape=jax.ShapeDtypeStruct((half, f), jnp.float32),
        in_specs=[
            pl.BlockSpec(memory_space=pltpu.VMEM),
            pl.BlockSpec(memory_space=pltpu.MemorySpace.HBM),
        ],
        out_specs=pl.BlockSpec(memory_space=pltpu.MemorySpace.HBM),
        scratch_shapes=[
            pltpu.VMEM((m, f), jnp.float32),
            pltpu.VMEM((2, 2 * qh, m), jnp.float32),
            pltpu.VMEM(qshape, jnp.bfloat16),
            pltpu.VMEM(qshape, jnp.bfloat16),
            pltpu.VMEM(qshape, jnp.bfloat16),
            pltpu.VMEM(qshape, jnp.bfloat16),
            pltpu.VMEM(qshape, jnp.bfloat16),
            pltpu.VMEM(qshape, jnp.bfloat16),
            pltpu.VMEM(qshape, jnp.float32),
            pltpu.VMEM(qshape, jnp.float32),
            pltpu.VMEM(qshape, jnp.float32),
            pltpu.VMEM(qshape, jnp.float32),
            pltpu.SemaphoreType.DMA,
            pltpu.SemaphoreType.DMA((4 * _C,)),
            pltpu.SemaphoreType.DMA((_C,)),
            pltpu.SemaphoreType.DMA((_C,)),
            pltpu.SemaphoreType.DMA((_C,)),
            pltpu.SemaphoreType.DMA((_C,)),
            pltpu.SemaphoreType.DMA((_C,)),
            pltpu.SemaphoreType.DMA((_C,)),
            pltpu.SemaphoreType.DMA((_C,)),
            pltpu.SemaphoreType.DMA((_C,)),
        ],
        compiler_params=pltpu.CompilerParams(collective_id=0),
    )(x, dy)


# device time: 18815 ns/iter; 1.0961x vs baseline; 1.0607x over previous
import jax
import jax.numpy as jnp
from jax import lax
from jax.experimental import pallas as pl
from jax.experimental.pallas import tpu as pltpu

_DIMS = (((1,), (0,)), ((), ()))
_C = 8


def kernel(x, dy):
    m, d = x.shape
    _, f = dy.shape
    half = d // 2
    zh = half // 2
    fc = f // _C

    def body(x_ref, dy_ref, out_ref, dyv, xt, ysend, yrecv, zsend, zrecv,
             zredf, zof, dy_sem, out_sem,
             ysend_sem, yrecv_sem, zsend_sem, zrecv_sem):
        my_x = lax.axis_index("x")
        my_y = lax.axis_index("y")
        my_z = lax.axis_index("z")
        ypartner = (my_x, 1 - my_y, my_z)
        zpartner = (my_x, my_y, 1 - my_z)

        dy_cp = pltpu.make_async_copy(dy_ref, dyv, dy_sem)
        dy_cp.start()

        barrier_sem = pltpu.get_barrier_semaphore()
        for nbr in (ypartner, zpartner):
            pl.semaphore_signal(
                barrier_sem, inc=1, device_id=nbr,
                device_id_type=pl.DeviceIdType.MESH,
            )
        pl.semaphore_wait(barrier_sem, 2)

        xt[0] = x_ref[:, pl.ds((1 - my_y) * half + my_z * zh, zh)].T
        xt[1] = x_ref[:, pl.ds(my_y * half + my_z * zh, zh)].T
        xs = xt[0]
        xo = xt[1]

        dy_cp.wait()

        y_rdmas = []
        for i in range(_C):
            ps = lax.dot_general(
                xs, dyv[:, i * fc:(i + 1) * fc], _DIMS,
                preferred_element_type=jnp.float32,
            )
            ysend[i] = ps.astype(jnp.bfloat16)
            r = pltpu.make_async_remote_copy(
                src_ref=ysend.at[i], dst_ref=yrecv.at[i],
                send_sem=ysend_sem.at[i], recv_sem=yrecv_sem.at[i],
                device_id=ypartner, device_id_type=pl.DeviceIdType.MESH,
            )
            r.start()
            y_rdmas.append(r)

        z_rdmas = []
        out_cps = []
        for i in range(_C):
            own = lax.dot_general(
                xo, dyv[:, i * fc:(i + 1) * fc], _DIMS,
                preferred_element_type=jnp.float32,
            )
            y_rdmas[i].wait_recv()
            red = own + yrecv[i].astype(jnp.float32)
            zredf[i] = red
            zsend[i] = red.astype(jnp.bfloat16)
            r = pltpu.make_async_remote_copy(
                src_ref=zsend.at[i], dst_ref=zrecv.at[i],
                send_sem=zsend_sem.at[i], recv_sem=zrecv_sem.at[i],
                device_id=zpartner, device_id_type=pl.DeviceIdType.MESH,
            )
            r.start()
            z_rdmas.append(r)
            cp = pltpu.make_async_copy(
                zredf.at[i],
                out_ref.at[pl.ds(my_z * zh, zh), pl.ds(i * fc, fc)],
                out_sem.at[i],
            )
            cp.start()
            out_cps.append(cp)

        for i in range(_C):
            z_rdmas[i].wait_recv()
            zof[i] = zrecv[i].astype(jnp.float32)
            cp = pltpu.make_async_copy(
                zof.at[i],
                out_ref.at[pl.ds((1 - my_z) * zh, zh), pl.ds(i * fc, fc)],
                out_sem.at[_C + i],
            )
            cp.start()
            out_cps.append(cp)

        for cp in out_cps:
            cp.wait()
        for i in range(_C):
            y_rdmas[i].wait_send()
            z_rdmas[i].wait_send()

    return pl.pallas_call(
        body,
        out_shape=jax.ShapeDtypeStruct((half, f), jnp.float32),
        in_specs=[
            pl.BlockSpec(memory_space=pltpu.VMEM),
            pl.BlockSpec(memory_space=pltpu.MemorySpace.HBM),
        ],
        out_specs=pl.BlockSpec(memory_space=pltpu.MemorySpace.HBM),
        scratch_shapes=[
            pltpu.VMEM((m, f), jnp.float32),
            pltpu.VMEM((2, zh, m), jnp.float32),
            pltpu.VMEM((_C, zh, fc), jnp.bfloat16),
            pltpu.VMEM((_C, zh, fc), jnp.bfloat16),
            pltpu.VMEM((_C, zh, fc), jnp.bfloat16),
            pltpu.VMEM((_C, zh, fc), jnp.bfloat16),
            pltpu.VMEM((_C, zh, fc), jnp.float32),
            pltpu.VMEM((_C, zh, fc), jnp.float32),
            pltpu.SemaphoreType.DMA,
            pltpu.SemaphoreType.DMA((2 * _C,)),
            pltpu.SemaphoreType.DMA((_C,)),
            pltpu.SemaphoreType.DMA((_C,)),
            pltpu.SemaphoreType.DMA((_C,)),
            pltpu.SemaphoreType.DMA((_C,)),
        ],
        compiler_params=pltpu.CompilerParams(collective_id=0),
    )(x, dy)
